# baseline (device time: 6408 ns/iter reference)
import jax
import jax.numpy as jnp
from jax import lax
from jax.experimental import pallas as pl
from jax.experimental.pallas import tpu as pltpu


def kernel(x, dy, gamma):
    del gamma
    m, d = x.shape

    def body(
        x_hbm, dy_hbm, out_ref,
        xv_ref, dyv_ref, part_ref, recv_ref,
        copy_sems, send_sem, recv_sem,
    ):
        my_x = lax.axis_index("x")
        my_y = lax.axis_index("y")
        my_z = lax.axis_index("z")
        partner = (1 - my_x, my_y, my_z)

        barrier_sem = pltpu.get_barrier_semaphore()
        pl.semaphore_signal(
            barrier_sem, inc=1, device_id=partner,
            device_id_type=pl.DeviceIdType.MESH,
        )

        n_chunk = 4
        mc = m // n_chunk
        copies = []
        for c in range(n_chunk):
            rows = pl.ds(c * mc, mc)
            cp_x = pltpu.make_async_copy(
                x_hbm.at[rows], xv_ref.at[rows], copy_sems.at[2 * c]
            )
            cp_dy = pltpu.make_async_copy(
                dy_hbm.at[rows], dyv_ref.at[rows], copy_sems.at[2 * c + 1]
            )
            cp_x.start()
            cp_dy.start()
            copies.append((cp_x, cp_dy))

        dgamma = None
        dbeta = None
        for c in range(n_chunk):
            cp_x, cp_dy = copies[c]
            cp_x.wait()
            cp_dy.wait()
            rows = pl.ds(c * mc, mc)
            xv = xv_ref[rows, :]
            dyv = dyv_ref[rows, :]
            mu = jnp.mean(xv, axis=1, keepdims=True)
            xc = xv - mu
            var = jnp.mean(xc * xc, axis=1, keepdims=True)
            rstd = lax.rsqrt(var + 1e-5)
            g = jnp.sum(dyv * (xc * rstd), axis=0)
            b = jnp.sum(dyv, axis=0)
            dgamma = g if dgamma is None else dgamma + g
            dbeta = b if dbeta is None else dbeta + b
        part_ref[0, :] = dgamma
        part_ref[1, :] = dbeta

        pl.semaphore_wait(barrier_sem, 1)

        rdma = pltpu.make_async_remote_copy(
            src_ref=part_ref,
            dst_ref=recv_ref,
            send_sem=send_sem,
            recv_sem=recv_sem,
            device_id=partner,
            device_id_type=pl.DeviceIdType.MESH,
        )
        rdma.start()
        rdma.wait()

        out_ref[:, :] = part_ref[:, :] + recv_ref[:, :]

    return pl.pallas_call(
        body,
        out_shape=jax.ShapeDtypeStruct((2, d), jnp.float32),
        in_specs=[
            pl.BlockSpec(memory_space=pltpu.MemorySpace.HBM),
            pl.BlockSpec(memory_space=pltpu.MemorySpace.HBM),
        ],
        out_specs=pl.BlockSpec(memory_space=pltpu.VMEM),
        scratch_shapes=[
            pltpu.VMEM((m, d), jnp.float32),
            pltpu.VMEM((m, d), jnp.float32),
            pltpu.VMEM((2, d), jnp.float32),
            pltpu.VMEM((2, d), jnp.float32),
            pltpu.SemaphoreType.DMA((8,)),
            pltpu.SemaphoreType.DMA,
            pltpu.SemaphoreType.DMA,
        ],
        compiler_params=pltpu.CompilerParams(collective_id=0),
    )(
        pltpu.with_memory_space_constraint(x, pltpu.MemorySpace.HBM),
        pltpu.with_memory_space_constraint(dy, pltpu.MemorySpace.HBM),
    )


# device time: 6225 ns/iter; 1.0294x vs baseline; 1.0294x over previous
import jax
import jax.numpy as jnp
from jax import lax
from jax.experimental import pallas as pl
from jax.experimental.pallas import tpu as pltpu


def kernel(x, dy, gamma):
    del gamma
    m, d = x.shape

    def body(
        x_hbm, dy_hbm, out_ref,
        xv_ref, dyv_ref, part_ref, recv_ref,
        copy_sems, send_sem, recv_sem,
    ):
        my_x = lax.axis_index("x")
        my_y = lax.axis_index("y")
        my_z = lax.axis_index("z")
        partner = (1 - my_x, my_y, my_z)

        barrier_sem = pltpu.get_barrier_semaphore()
        pl.semaphore_signal(
            barrier_sem, inc=1, device_id=partner,
            device_id_type=pl.DeviceIdType.MESH,
        )

        n_chunk = 4
        mc = m // n_chunk
        copies = []
        for c in range(n_chunk):
            rows = pl.ds(c * mc, mc)
            cp_x = pltpu.make_async_copy(
                x_hbm.at[rows], xv_ref.at[rows], copy_sems.at[2 * c]
            )
            cp_dy = pltpu.make_async_copy(
                dy_hbm.at[rows], dyv_ref.at[rows], copy_sems.at[2 * c + 1]
            )
            cp_x.start()
            cp_dy.start()
            copies.append((cp_x, cp_dy))

        dgamma = None
        dbeta = None
        for c in range(n_chunk):
            cp_x, cp_dy = copies[c]
            cp_x.wait()
            cp_dy.wait()
            rows = pl.ds(c * mc, mc)
            xv = xv_ref[rows, :]
            dyv = dyv_ref[rows, :]
            s1 = jnp.sum(xv, axis=1, keepdims=True)
            s2 = jnp.sum(xv * xv, axis=1, keepdims=True)
            mu = s1 * (1.0 / d)
            var = s2 * (1.0 / d) - mu * mu
            rstd = lax.rsqrt(var + 1e-5)
            g = jnp.sum(dyv * xv * rstd - dyv * (mu * rstd), axis=0)
            b = jnp.sum(dyv, axis=0)
            dgamma = g if dgamma is None else dgamma + g
            dbeta = b if dbeta is None else dbeta + b
        part_ref[0, :] = dgamma
        part_ref[1, :] = dbeta

        pl.semaphore_wait(barrier_sem, 1)

        rdma = pltpu.make_async_remote_copy(
            src_ref=part_ref,
            dst_ref=recv_ref,
            send_sem=send_sem,
            recv_sem=recv_sem,
            device_id=partner,
            device_id_type=pl.DeviceIdType.MESH,
        )
        rdma.start()
        rdma.wait()

        part_ref[:, :] = part_ref[:, :] + recv_ref[:, :]
        cp_out = pltpu.make_async_copy(part_ref, out_ref, copy_sems.at[0])
        cp_out.start()
        cp_out.wait()

    return pl.pallas_call(
        body,
        out_shape=jax.ShapeDtypeStruct((2, d), jnp.float32),
        in_specs=[
            pl.BlockSpec(memory_space=pltpu.MemorySpace.HBM),
            pl.BlockSpec(memory_space=pltpu.MemorySpace.HBM),
        ],
        out_specs=pl.BlockSpec(memory_space=pltpu.MemorySpace.HBM),
        scratch_shapes=[
            pltpu.VMEM((m, d), jnp.float32),
            pltpu.VMEM((m, d), jnp.float32),
            pltpu.VMEM((2, d), jnp.float32),
            pltpu.VMEM((2, d), jnp.float32),
            pltpu.SemaphoreType.DMA((8,)),
            pltpu.SemaphoreType.DMA,
            pltpu.SemaphoreType.DMA,
        ],
        compiler_params=pltpu.CompilerParams(collective_id=0),
    )(
        pltpu.with_memory_space_constraint(x, pltpu.MemorySpace.HBM),
        pltpu.with_memory_space_constraint(dy, pltpu.MemorySpace.HBM),
    )


# device time: 6140 ns/iter; 1.0436x vs baseline; 1.0138x over previous
import jax
import jax.numpy as jnp
from jax import lax
from jax.experimental import pallas as pl
from jax.experimental.pallas import tpu as pltpu


def kernel(x, dy, gamma):
    del gamma
    m, d = x.shape

    def body(
        x_hbm, dy_hbm, out_ref,
        xv_ref, dyv_ref, part_ref, recv_ref,
        copy_sems, send_sem, recv_sem,
    ):
        my_x = lax.axis_index("x")
        my_y = lax.axis_index("y")
        my_z = lax.axis_index("z")
        partner = (1 - my_x, my_y, my_z)

        barrier_sem = pltpu.get_barrier_semaphore()
        pl.semaphore_signal(
            barrier_sem, inc=1, device_id=partner,
            device_id_type=pl.DeviceIdType.MESH,
        )

        n_chunk = 4
        mc = m // n_chunk
        copies = []
        for c in range(n_chunk):
            rows = pl.ds(c * mc, mc)
            cp_x = pltpu.make_async_copy(
                x_hbm.at[rows], xv_ref.at[rows], copy_sems.at[2 * c]
            )
            cp_dy = pltpu.make_async_copy(
                dy_hbm.at[rows], dyv_ref.at[rows], copy_sems.at[2 * c + 1]
            )
            cp_x.start()
            cp_dy.start()
            copies.append((cp_x, cp_dy))

        dgamma = None
        dbeta = None
        for c in range(n_chunk):
            cp_x, cp_dy = copies[c]
            cp_x.wait()
            cp_dy.wait()
            rows = pl.ds(c * mc, mc)
            xv = xv_ref[rows, :]
            dyv = dyv_ref[rows, :]
            s1 = jnp.sum(xv, axis=1, keepdims=True)
            s2 = jnp.sum(xv * xv, axis=1, keepdims=True)
            mu = s1 * (1.0 / d)
            var = s2 * (1.0 / d) - mu * mu
            rstd = lax.rsqrt(var + 1e-5)
            g = jnp.sum(dyv * (xv * rstd - mu * rstd), axis=0)
            b = jnp.sum(dyv, axis=0)
            dgamma = g if dgamma is None else dgamma + g
            dbeta = b if dbeta is None else dbeta + b
        part_ref[0, :] = dgamma
        part_ref[1, :] = dbeta

        pl.semaphore_wait(barrier_sem, 1)

        rdma = pltpu.make_async_remote_copy(
            src_ref=part_ref,
            dst_ref=recv_ref,
            send_sem=send_sem,
            recv_sem=recv_sem,
            device_id=partner,
            device_id_type=pl.DeviceIdType.MESH,
        )
        rdma.start()
        rdma.wait_recv()

        recv_ref[:, :] = part_ref[:, :] + recv_ref[:, :]
        cp_out = pltpu.make_async_copy(recv_ref, out_ref, copy_sems.at[0])
        cp_out.start()
        rdma.wait_send()
        cp_out.wait()

    return pl.pallas_call(
        body,
        out_shape=jax.ShapeDtypeStruct((2, d), jnp.float32),
        in_specs=[
            pl.BlockSpec(memory_space=pltpu.MemorySpace.HBM),
            pl.BlockSpec(memory_space=pltpu.MemorySpace.HBM),
        ],
        out_specs=pl.BlockSpec(memory_space=pl.MemorySpace.ANY),
        scratch_shapes=[
            pltpu.VMEM((m, d), jnp.float32),
            pltpu.VMEM((m, d), jnp.float32),
            pltpu.VMEM((2, d), jnp.float32),
            pltpu.VMEM((2, d), jnp.float32),
            pltpu.SemaphoreType.DMA((8,)),
            pltpu.SemaphoreType.DMA,
            pltpu.SemaphoreType.DMA,
        ],
        compiler_params=pltpu.CompilerParams(collective_id=0),
    )(
        pltpu.with_memory_space_constraint(x, pltpu.MemorySpace.HBM),
        pltpu.with_memory_space_constraint(dy, pltpu.MemorySpace.HBM),
    )
